# baseline (speedup 1.0000x reference)
import functools

import jax
import jax.numpy as jnp
from jax import lax
from jax.experimental import pallas as pl
from jax.experimental.pallas import tpu as pltpu
from jax.experimental.pallas import tpu_sc as plsc

CUTOFF = 8.0
NUM_WORKERS = 32
LANES = 16


def _sc_pair_dist2(xs, ys, zs, idx_i, idx_j):
    n_atoms = xs.shape[0]
    p = idx_i.shape[0]
    per_w = -(-p // NUM_WORKERS)
    per_w = -(-per_w // LANES) * LANES
    chunks = per_w // LANES
    mesh = plsc.VectorSubcoreMesh(core_axis_name="c", subcore_axis_name="s")

    @functools.partial(
        pl.kernel,
        out_type=jax.ShapeDtypeStruct((p,), jnp.float32),
        mesh=mesh,
        scratch_types=[
            pltpu.VMEM((n_atoms,), jnp.float32),
            pltpu.VMEM((per_w,), jnp.int32),
            pltpu.VMEM((per_w,), jnp.int32),
            pltpu.VMEM((per_w,), jnp.float32),
        ],
        compiler_params=pltpu.CompilerParams(needs_layout_passes=False),
    )
    def sc_kernel(x_hbm, y_hbm, z_hbm, ii_hbm, jj_hbm, d2_hbm,
                  tab_v, ii_v, jj_v, d2_v):
        wid = lax.axis_index("s") * 2 + lax.axis_index("c")
        base = jnp.minimum(wid * per_w, p - per_w)
        pltpu.sync_copy(ii_hbm.at[pl.ds(base, per_w)], ii_v)
        pltpu.sync_copy(jj_hbm.at[pl.ds(base, per_w)], jj_v)
        for c, coord_hbm in enumerate((x_hbm, y_hbm, z_hbm)):
            pltpu.sync_copy(coord_hbm, tab_v)
            first = c == 0

            @plsc.parallel_loop(0, per_w, step=LANES, unroll=8)
            def _(off, _first=first):
                ii = ii_v[pl.ds(off, LANES)]
                jj = jj_v[pl.ds(off, LANES)]
                xi = plsc.load_gather(tab_v, [ii])
                xj = plsc.load_gather(tab_v, [jj])
                d = xj - xi
                if _first:
                    d2_v[pl.ds(off, LANES)] = d * d
                else:
                    d2_v[pl.ds(off, LANES)] = d2_v[pl.ds(off, LANES)] + d * d
        pltpu.sync_copy(d2_v, d2_hbm.at[pl.ds(base, per_w)])

    return sc_kernel(xs, ys, zs, idx_i, idx_j)


def _sc_features(ids, table):
    n = ids.shape[0]
    v, d = table.shape
    per_w = 1600
    n_chunks = 4
    chunk = per_w // n_chunks
    mesh = plsc.VectorSubcoreMesh(core_axis_name="c", subcore_axis_name="s")

    @functools.partial(
        pl.kernel,
        out_type=jax.ShapeDtypeStruct((n, d), jnp.float32),
        mesh=mesh,
        scratch_types=[
            pltpu.VMEM((per_w,), jnp.int32),
            pltpu.VMEM((chunk, d), jnp.float32),
            pltpu.SemaphoreType.DMA,
        ],
        compiler_params=pltpu.CompilerParams(needs_layout_passes=False),
    )
    def feat_kernel(ids_hbm, tab_hbm, out_hbm, ids_v, rows_v, sem):
        wid = lax.axis_index("s") * 2 + lax.axis_index("c")
        base = jnp.minimum(wid * per_w, n - per_w)
        pltpu.sync_copy(ids_hbm.at[pl.ds(base, per_w)], ids_v)
        for k in range(n_chunks):
            pltpu.async_copy(
                tab_hbm.at[ids_v.at[pl.ds(k * chunk, chunk)]],
                rows_v, sem).wait()
            pltpu.sync_copy(rows_v, out_hbm.at[pl.ds(base + k * chunk, chunk), :])

    return feat_kernel(ids, table)


def _tc_rbf(d2_sq, centers_row, widths_row, n_rbf):
    rows, w128 = d2_sq.shape
    r_blk = 256
    grid = -(-rows // r_blk)

    def body(d2_ref, c_ref, w_ref, d_ref, cut_ref, rbf_ref):
        d2 = d2_ref[...]
        dd = jnp.sqrt(d2)
        d_ref[...] = dd
        x = dd * (1.0 / CUTOFF)
        x3 = x * x * x
        f = 1.0 + x3 * (-10.0 + x * (15.0 - 6.0 * x))
        cut_ref[...] = jnp.where(dd < CUTOFF, f, jnp.zeros_like(f))
        c_col = c_ref[...]
        w_col = w_ref[...]
        pieces = []
        for r in range(r_blk):
            z = dd[r:r + 1, :] - c_col
            pieces.append(jnp.exp(-w_col * z * z))
        rbf_ref[...] = jnp.concatenate(pieces, axis=1)

    return pl.pallas_call(
        body,
        grid=(grid,),
        in_specs=[
            pl.BlockSpec((r_blk, w128), lambda i: (i, 0)),
            pl.BlockSpec((n_rbf, 1), lambda i: (0, 0)),
            pl.BlockSpec((n_rbf, 1), lambda i: (0, 0)),
        ],
        out_specs=[
            pl.BlockSpec((r_blk, w128), lambda i: (i, 0)),
            pl.BlockSpec((r_blk, w128), lambda i: (i, 0)),
            pl.BlockSpec((n_rbf, r_blk * w128), lambda i: (0, i)),
        ],
        out_shape=[
            jax.ShapeDtypeStruct((rows, w128), jnp.float32),
            jax.ShapeDtypeStruct((rows, w128), jnp.float32),
            jax.ShapeDtypeStruct((n_rbf, rows * w128), jnp.float32),
        ],
    )(d2_sq, centers_row, widths_row)


def kernel(atomic_numbers, positions, idx_i, idx_j,
           atom_features, rbf_centers, rbf_widths):
    p = idx_i.shape[0]
    n_rbf = rbf_centers.shape[0]

    ii = idx_i.astype(jnp.int32)
    jj = idx_j.astype(jnp.int32)
    pos = positions.astype(jnp.float32)
    d2 = _sc_pair_dist2(pos[:, 0], pos[:, 1], pos[:, 2], ii, jj)

    d_sq, cut_sq, rbfs_t = _tc_rbf(
        d2.reshape(p // 128, 128),
        rbf_centers.astype(jnp.float32).reshape(n_rbf, 1),
        rbf_widths.astype(jnp.float32).reshape(n_rbf, 1),
        n_rbf,
    )
    distances = d_sq.reshape(p)
    cutoffs = cut_sq.reshape(p)
    rbfs = jnp.transpose(rbfs_t)

    features = _sc_features(atomic_numbers.astype(jnp.int32),
                            atom_features.astype(jnp.float32))

    return (features, distances, cutoffs, rbfs, distances)


# speedup vs baseline: 1.0262x; 1.0262x over previous
import functools

import jax
import jax.numpy as jnp
from jax import lax
from jax.experimental import pallas as pl
from jax.experimental.pallas import tpu as pltpu
from jax.experimental.pallas import tpu_sc as plsc

CUTOFF = 8.0
NUM_WORKERS = 32
LANES = 16


def _sc_pair_dist2(xs, ys, zs, idx_i, idx_j):
    n_atoms = xs.shape[0]
    p = idx_i.shape[0]
    per_w = -(-p // NUM_WORKERS)
    per_w = -(-per_w // LANES) * LANES
    chunks = per_w // LANES
    mesh = plsc.VectorSubcoreMesh(core_axis_name="c", subcore_axis_name="s")

    @functools.partial(
        pl.kernel,
        out_type=jax.ShapeDtypeStruct((p,), jnp.float32),
        mesh=mesh,
        scratch_types=[
            pltpu.VMEM((n_atoms,), jnp.float32),
            pltpu.VMEM((per_w,), jnp.int32),
            pltpu.VMEM((per_w,), jnp.int32),
            pltpu.VMEM((per_w,), jnp.float32),
        ],
        compiler_params=pltpu.CompilerParams(needs_layout_passes=False),
    )
    def sc_kernel(x_hbm, y_hbm, z_hbm, ii_hbm, jj_hbm, d2_hbm,
                  tab_v, ii_v, jj_v, d2_v):
        wid = lax.axis_index("s") * 2 + lax.axis_index("c")
        base = jnp.minimum(wid * per_w, p - per_w)
        pltpu.sync_copy(ii_hbm.at[pl.ds(base, per_w)], ii_v)
        pltpu.sync_copy(jj_hbm.at[pl.ds(base, per_w)], jj_v)
        for c, coord_hbm in enumerate((x_hbm, y_hbm, z_hbm)):
            pltpu.sync_copy(coord_hbm, tab_v)
            first = c == 0

            @plsc.parallel_loop(0, per_w, step=LANES, unroll=8)
            def _(off, _first=first):
                ii = ii_v[pl.ds(off, LANES)]
                jj = jj_v[pl.ds(off, LANES)]
                xi = plsc.load_gather(tab_v, [ii])
                xj = plsc.load_gather(tab_v, [jj])
                d = xj - xi
                if _first:
                    d2_v[pl.ds(off, LANES)] = d * d
                else:
                    d2_v[pl.ds(off, LANES)] = d2_v[pl.ds(off, LANES)] + d * d
        pltpu.sync_copy(d2_v, d2_hbm.at[pl.ds(base, per_w)])

    return sc_kernel(xs, ys, zs, idx_i, idx_j)


def _sc_features(ids, table):
    n = ids.shape[0]
    v, d = table.shape
    per_w = 1600
    n_chunks = 4
    chunk = per_w // n_chunks
    mesh = plsc.VectorSubcoreMesh(core_axis_name="c", subcore_axis_name="s")

    @functools.partial(
        pl.kernel,
        out_type=jax.ShapeDtypeStruct((n, d), jnp.float32),
        mesh=mesh,
        scratch_types=[
            pltpu.VMEM((per_w,), jnp.int32),
            pltpu.VMEM((chunk, d), jnp.float32),
            pltpu.VMEM((chunk, d), jnp.float32),
            pltpu.SemaphoreType.DMA,
            pltpu.SemaphoreType.DMA,
        ],
        compiler_params=pltpu.CompilerParams(needs_layout_passes=False),
    )
    def feat_kernel(ids_hbm, tab_hbm, out_hbm, ids_v, rows0, rows1, sem0, sem1):
        wid = lax.axis_index("s") * 2 + lax.axis_index("c")
        base = jnp.minimum(wid * per_w, n - per_w)
        pltpu.sync_copy(ids_hbm.at[pl.ds(base, per_w)], ids_v)
        bufs = (rows0, rows1)
        sems = (sem0, sem1)

        def gather(k):
            return pltpu.async_copy(
                tab_hbm.at[ids_v.at[pl.ds(k * chunk, chunk)]],
                bufs[k % 2], sems[k % 2])

        cp = gather(0)
        for k in range(n_chunks):
            cp.wait()
            if k + 1 < n_chunks:
                cp = gather(k + 1)
            pltpu.sync_copy(bufs[k % 2],
                            out_hbm.at[pl.ds(base + k * chunk, chunk), :])

    return feat_kernel(ids, table)


def _tc_rbf(d2_sq, centers_row, widths_row, n_rbf):
    rows, w128 = d2_sq.shape
    r_blk = 128
    grid = -(-rows // r_blk)

    def body(d2_ref, c_ref, w_ref, d_ref, cut_ref, rbf_ref):
        d2 = d2_ref[...]
        dd = jnp.sqrt(d2)
        d_ref[...] = dd
        x = dd * (1.0 / CUTOFF)
        x3 = x * x * x
        f = 1.0 + x3 * (-10.0 + x * (15.0 - 6.0 * x))
        cut_ref[...] = jnp.where(dd < CUTOFF, f, jnp.zeros_like(f))
        c_col = c_ref[...]
        w_col = w_ref[...]
        pieces = []
        for r in range(r_blk):
            z = dd[r:r + 1, :] - c_col
            pieces.append(jnp.exp(-w_col * z * z))
        rbf_ref[...] = jnp.concatenate(pieces, axis=1)

    return pl.pallas_call(
        body,
        grid=(grid,),
        in_specs=[
            pl.BlockSpec((r_blk, w128), lambda i: (i, 0)),
            pl.BlockSpec((n_rbf, 1), lambda i: (0, 0)),
            pl.BlockSpec((n_rbf, 1), lambda i: (0, 0)),
        ],
        out_specs=[
            pl.BlockSpec((r_blk, w128), lambda i: (i, 0)),
            pl.BlockSpec((r_blk, w128), lambda i: (i, 0)),
            pl.BlockSpec((n_rbf, r_blk * w128), lambda i: (0, i)),
        ],
        out_shape=[
            jax.ShapeDtypeStruct((rows, w128), jnp.float32),
            jax.ShapeDtypeStruct((rows, w128), jnp.float32),
            jax.ShapeDtypeStruct((n_rbf, rows * w128), jnp.float32),
        ],
    )(d2_sq, centers_row, widths_row)


def kernel(atomic_numbers, positions, idx_i, idx_j,
           atom_features, rbf_centers, rbf_widths):
    p = idx_i.shape[0]
    n_rbf = rbf_centers.shape[0]

    ii = idx_i.astype(jnp.int32)
    jj = idx_j.astype(jnp.int32)
    pos = positions.astype(jnp.float32)
    d2 = _sc_pair_dist2(pos[:, 0], pos[:, 1], pos[:, 2], ii, jj)

    d_sq, cut_sq, rbfs_t = _tc_rbf(
        d2.reshape(p // 128, 128),
        rbf_centers.astype(jnp.float32).reshape(n_rbf, 1),
        rbf_widths.astype(jnp.float32).reshape(n_rbf, 1),
        n_rbf,
    )
    distances = d_sq.reshape(p)
    cutoffs = cut_sq.reshape(p)
    rbfs = jnp.transpose(rbfs_t)

    features = _sc_features(atomic_numbers.astype(jnp.int32),
                            atom_features.astype(jnp.float32))

    return (features, distances, cutoffs, rbfs, distances)
